# baseline (speedup 1.0000x reference)
import jax
import jax.numpy as jnp
from jax.experimental import pallas as pl
from jax.experimental.pallas import tpu as pltpu

DIM = 128
NUM_HEADS = 8
N_WIN = 7
TOPK = 4
HEAD_DIM = DIM // NUM_HEADS
SCALE = DIM ** -0.5
NREG = N_WIN ** 3
RSS = 64
SEQ = NREG * RSS
SLAB = 4 * 28 * 28
HBLK = 1568

_INTERPRET = False

_QKV_RBLK = 49


def _qkvroute_kernel(x_ref, w_ref, b_ref, q_ref, k_ref, v_ref, idx_ref,
                     pool_ref):
    i = pl.program_id(0)

    @pl.when(i < 7)
    def _qkv():
        y = jnp.dot(x_ref[:], w_ref[:],
                    preferred_element_type=jnp.float32) + b_ref[:]
        q_ref[:] = y[:, :DIM]
        k_ref[:] = y[:, DIM:2 * DIM]
        v_ref[:] = y[:, 2 * DIM:]
        p = y[:, :2 * DIM].reshape(_QKV_RBLK, RSS, 2 * DIM)
        pool_ref[i] = jnp.sum(p, axis=1) * (1.0 / RSS)

    @pl.when(i == 7)
    def _route():
        pools = pool_ref[:].reshape(NREG, 2 * DIM)
        qp = pools[:, :DIM]
        kp = pools[:, DIM:]
        a = jax.lax.dot_general(qp, kp, (((1,), (1,)), ((), ())),
                                preferred_element_type=jnp.float32)
        col = jax.lax.broadcasted_iota(jnp.int32, a.shape, 1)
        for j in range(TOPK):
            m = jnp.max(a, axis=1, keepdims=True)
            cand = jnp.where(a >= m, col, NREG + 1)
            sel = jnp.min(cand, axis=1, keepdims=True)
            idx_ref[:, j:j + 1] = sel
            a = jnp.where(col == sel, -jnp.inf, a)


def _qkvroute_call(x_seq, w_t, b2d):
    m_blk = _QKV_RBLK * RSS
    seq_spec = pl.BlockSpec((m_blk, DIM), lambda i: (jnp.clip(i, 0, 6), 0))
    return pl.pallas_call(
        _qkvroute_kernel,
        grid=(8,),
        in_specs=[
            seq_spec,
            pl.BlockSpec((DIM, 3 * DIM), lambda i: (0, 0)),
            pl.BlockSpec((1, 3 * DIM), lambda i: (0, 0)),
        ],
        out_specs=[
            seq_spec, seq_spec, seq_spec,
            pl.BlockSpec((NREG, TOPK), lambda i: (0, 0)),
        ],
        out_shape=[
            jax.ShapeDtypeStruct((SEQ, DIM), jnp.float32),
            jax.ShapeDtypeStruct((SEQ, DIM), jnp.float32),
            jax.ShapeDtypeStruct((SEQ, DIM), jnp.float32),
            jax.ShapeDtypeStruct((NREG, TOPK), jnp.int32),
        ],
        scratch_shapes=[pltpu.VMEM((7, _QKV_RBLK, 2 * DIM), jnp.float32)],
        interpret=_INTERPRET,
    )(x_seq, w_t, b2d)


def _attn_kernel(idx_ref, q_ref, k_ref, v_ref, o_ref):
    r = pl.program_id(0)
    q = q_ref[:] * SCALE
    ks = [k_ref[pl.ds(idx_ref[r, j] * RSS, RSS), :] for j in range(TOPK)]
    vs = [v_ref[pl.ds(idx_ref[r, j] * RSS, RSS), :] for j in range(TOPK)]
    kg = jnp.concatenate(ks, axis=0)
    vg = jnp.concatenate(vs, axis=0)
    big = NUM_HEADS * RSS
    qt = jnp.broadcast_to(q[None], (NUM_HEADS, RSS, DIM)).reshape(big, DIM)
    rowh = jax.lax.broadcasted_iota(jnp.int32, (big, DIM), 0) // RSS
    colh = jax.lax.broadcasted_iota(jnp.int32, (big, DIM), 1) // HEAD_DIM
    qbd = jnp.where(rowh == colh, qt, 0.0).astype(jnp.bfloat16)
    s = jax.lax.dot_general(qbd, kg.astype(jnp.bfloat16),
                            (((1,), (1,)), ((), ())),
                            preferred_element_type=jnp.float32)
    s = s - jnp.max(s, axis=1, keepdims=True)
    e = jnp.exp(s)
    denom = jnp.sum(e, axis=1, keepdims=True)
    o3 = jnp.dot(e.astype(jnp.bfloat16), vg.astype(jnp.bfloat16),
                 preferred_element_type=jnp.float32)
    o3 = o3 / denom
    hsel = jax.lax.broadcasted_iota(jnp.int32, (RSS, DIM), 1) // HEAD_DIM
    acc = jnp.zeros((RSS, DIM), jnp.float32)
    for m in range(NUM_HEADS):
        acc = acc + jnp.where(hsel == m, o3[m * RSS:(m + 1) * RSS, :], 0.0)
    o_ref[:] = acc


def _attn_call(q, k, v, idx):
    return pl.pallas_call(
        _attn_kernel,
        grid=(NREG,),
        in_specs=[
            pl.BlockSpec(memory_space=pltpu.SMEM),
            pl.BlockSpec((RSS, DIM), lambda r: (r, 0)),
            pl.BlockSpec((SEQ, DIM), lambda r: (0, 0)),
            pl.BlockSpec((SEQ, DIM), lambda r: (0, 0)),
        ],
        out_specs=pl.BlockSpec((RSS, DIM), lambda r: (r, 0)),
        out_shape=jax.ShapeDtypeStruct((SEQ, DIM), jnp.float32),
        interpret=_INTERPRET,
    )(idx, q, k, v)


def _fin_kernel(v0_ref, v1_ref, v2_ref, v3_ref, at_ref, w_ref, bl_ref,
                wo_ref, bo_ref, o_ref):
    a = pl.program_id(0)
    vp = jnp.concatenate([v0_ref[:], v1_ref[:], v2_ref[:], v3_ref[:]], axis=0)
    n = jax.lax.broadcasted_iota(jnp.int32, (SLAB, 1), 0)
    h_pos = 4 * a + n // 784
    w_pos = (n // 28) % 28
    d_pos = n % 28
    hmask = {-1: (h_pos > 0), 0: None, 1: (h_pos < 27)}
    wmask = {-1: (w_pos > 0), 0: None, 1: (w_pos < 27)}
    dmask = {-1: (d_pos > 0), 0: None, 1: (d_pos < 27)}
    acc = at_ref[:] + bl_ref[:]
    for t in range(27):
        dh, dw, du = t // 9 - 1, (t // 3) % 3 - 1, t % 3 - 1
        delta = 784 * dh + 28 * dw + du
        sl = vp[HBLK + delta:HBLK + delta + SLAB, :]
        m = None
        for part in (hmask[dh], wmask[dw], dmask[du]):
            if part is not None:
                m = part if m is None else (m & part)
        if m is None:
            acc = acc + sl * w_ref[t:t + 1, :]
        else:
            acc = acc + sl * w_ref[t:t + 1, :] * m.astype(jnp.float32)
    o_ref[:] = jnp.dot(acc, wo_ref[:],
                       preferred_element_type=jnp.float32) + bo_ref[:]


def _fin_call(v_flat, attn_flat, w27, bl2d, wo, bo_col):
    nblk = SEQ // HBLK - 1
    return pl.pallas_call(
        _fin_kernel,
        grid=(N_WIN,),
        in_specs=[
            pl.BlockSpec((HBLK, DIM), lambda a: (jnp.clip(2 * a - 1, 0, 13), 0)),
            pl.BlockSpec((HBLK, DIM), lambda a: (2 * a, 0)),
            pl.BlockSpec((HBLK, DIM), lambda a: (2 * a + 1, 0)),
            pl.BlockSpec((HBLK, DIM), lambda a: (jnp.clip(2 * a + 2, 0, 13), 0)),
            pl.BlockSpec((SLAB, DIM), lambda a: (a, 0)),
            pl.BlockSpec((27, DIM), lambda a: (0, 0)),
            pl.BlockSpec((1, DIM), lambda a: (0, 0)),
            pl.BlockSpec((DIM, DIM), lambda a: (0, 0)),
            pl.BlockSpec((1, DIM), lambda a: (0, 0)),
        ],
        out_specs=pl.BlockSpec((SLAB, DIM), lambda a: (a, 0)),
        out_shape=jax.ShapeDtypeStruct((SEQ, DIM), jnp.float32),
        interpret=_INTERPRET,
    )(v_flat, v_flat, v_flat, v_flat, attn_flat, w27, bl2d, wo, bo_col)


def kernel(x, W_qkv, b_qkv, W_lepe, b_lepe, W_out, b_out):
    C, H, W_, D = DIM, 28, 28, 28
    rs = H // N_WIN
    xt = x[0].reshape(C, N_WIN, rs, N_WIN, rs, N_WIN, rs)
    xt = jnp.transpose(xt, (1, 3, 5, 2, 4, 6, 0)).reshape(SEQ, C)

    q, k, v, idx = _qkvroute_call(xt, W_qkv.T, b_qkv[None, :])
    attn_seq = _attn_call(q, k, v, idx)

    def seq2grid_flat(t):
        t = t.reshape(N_WIN, N_WIN, N_WIN, rs, rs, rs, C)
        t = jnp.transpose(t, (0, 3, 1, 4, 2, 5, 6))
        return t.reshape(SEQ, C)

    out_flat = _fin_call(seq2grid_flat(v), seq2grid_flat(attn_seq),
                         W_lepe.reshape(C, 27).T, b_lepe[None, :],
                         W_out.T, b_out[None, :])
    out = jnp.transpose(out_flat.reshape(H, W_, D, C), (3, 0, 1, 2))
    return out[None]


# speedup vs baseline: 1.1344x; 1.1344x over previous
import jax
import jax.numpy as jnp
from jax.experimental import pallas as pl
from jax.experimental.pallas import tpu as pltpu

DIM = 128
NUM_HEADS = 8
N_WIN = 7
TOPK = 4
HEAD_DIM = DIM // NUM_HEADS
SCALE = DIM ** -0.5
NREG = N_WIN ** 3
RSS = 64
SEQ = NREG * RSS
PSEQ = 7 * 7 * 8 * RSS
PSLAB = 4 * 28 * 32
HBLK = PSLAB // 2

_INTERPRET = False

_QKV_RBLK = 49


def _qkvroute_kernel(x_ref, w_ref, b_ref, q_ref, k_ref, v_ref, idx_ref,
                     pool_ref):
    i = pl.program_id(0)

    @pl.when(i < 7)
    def _qkv():
        y = jnp.dot(x_ref[:], w_ref[:],
                    preferred_element_type=jnp.float32) + b_ref[:]
        q_ref[:] = y[:, :DIM]
        k_ref[:] = y[:, DIM:2 * DIM]
        yv = y[:, 2 * DIM:]
        for b in range(7):
            v_ref[b * 512:b * 512 + 448, :] = yv[b * 448:(b + 1) * 448, :]
            v_ref[b * 512 + 448:(b + 1) * 512, :] = jnp.zeros((64, DIM),
                                                              jnp.float32)
        p = y[:, :2 * DIM].reshape(_QKV_RBLK, RSS, 2 * DIM)
        pool_ref[i] = jnp.sum(p, axis=1) * (1.0 / RSS)

    @pl.when(i == 7)
    def _route():
        pools = pool_ref[:].reshape(NREG, 2 * DIM)
        qp = pools[:, :DIM]
        kp = pools[:, DIM:]
        a = jax.lax.dot_general(qp, kp, (((1,), (1,)), ((), ())),
                                preferred_element_type=jnp.float32)
        col = jax.lax.broadcasted_iota(jnp.int32, a.shape, 1)
        for j in range(TOPK):
            m = jnp.max(a, axis=1, keepdims=True)
            cand = jnp.where(a >= m, col, NREG + 1)
            sel = jnp.min(cand, axis=1, keepdims=True)
            idx_ref[:, j:j + 1] = sel
            a = jnp.where(col == sel, -jnp.inf, a)


def _qkvroute_call(x_seq, w_t, b2d):
    m_blk = _QKV_RBLK * RSS
    seq_spec = pl.BlockSpec((m_blk, DIM), lambda i: (jnp.clip(i, 0, 6), 0))
    return pl.pallas_call(
        _qkvroute_kernel,
        grid=(8,),
        in_specs=[
            seq_spec,
            pl.BlockSpec((DIM, 3 * DIM), lambda i: (0, 0)),
            pl.BlockSpec((1, 3 * DIM), lambda i: (0, 0)),
        ],
        out_specs=[
            seq_spec, seq_spec,
            pl.BlockSpec((PSLAB, DIM), lambda i: (jnp.clip(i, 0, 6), 0)),
            pl.BlockSpec((NREG, TOPK), lambda i: (0, 0)),
        ],
        out_shape=[
            jax.ShapeDtypeStruct((SEQ, DIM), jnp.float32),
            jax.ShapeDtypeStruct((SEQ, DIM), jnp.float32),
            jax.ShapeDtypeStruct((PSEQ, DIM), jnp.float32),
            jax.ShapeDtypeStruct((NREG, TOPK), jnp.int32),
        ],
        scratch_shapes=[pltpu.VMEM((7, _QKV_RBLK, 2 * DIM), jnp.float32)],
        interpret=_INTERPRET,
    )(x_seq, w_t, b2d)


def _attn_kernel(idx_ref, q_ref, k_ref, v_ref, o_ref):
    r = pl.program_id(0)
    q = q_ref[:] * SCALE
    ks, vs = [], []
    for j in range(TOPK):
        g = idx_ref[r, j]
        ks.append(k_ref[pl.ds(g * RSS, RSS), :])
        vs.append(v_ref[pl.ds((g + g // 7) * RSS, RSS), :])
    kg = jnp.concatenate(ks, axis=0)
    vg = jnp.concatenate(vs, axis=0)
    big = NUM_HEADS * RSS
    qt = jnp.broadcast_to(q[None], (NUM_HEADS, RSS, DIM)).reshape(big, DIM)
    rowh = jax.lax.broadcasted_iota(jnp.int32, (big, DIM), 0) // RSS
    colh = jax.lax.broadcasted_iota(jnp.int32, (big, DIM), 1) // HEAD_DIM
    qbd = jnp.where(rowh == colh, qt, 0.0).astype(jnp.bfloat16)
    s = jax.lax.dot_general(qbd, kg.astype(jnp.bfloat16),
                            (((1,), (1,)), ((), ())),
                            preferred_element_type=jnp.float32)
    s = s - jnp.max(s, axis=1, keepdims=True)
    e = jnp.exp(s)
    denom = jnp.sum(e, axis=1, keepdims=True)
    o3 = jnp.dot(e.astype(jnp.bfloat16), vg.astype(jnp.bfloat16),
                 preferred_element_type=jnp.float32)
    o3 = o3 / denom
    hsel = jax.lax.broadcasted_iota(jnp.int32, (RSS, DIM), 1) // HEAD_DIM
    acc = jnp.zeros((RSS, DIM), jnp.float32)
    for m in range(NUM_HEADS):
        acc = acc + jnp.where(hsel == m, o3[m * RSS:(m + 1) * RSS, :], 0.0)
    o_ref[:] = acc


def _attn_call(q, k, v, idx):
    return pl.pallas_call(
        _attn_kernel,
        grid=(NREG,),
        in_specs=[
            pl.BlockSpec(memory_space=pltpu.SMEM),
            pl.BlockSpec((RSS, DIM), lambda r: (r, 0)),
            pl.BlockSpec((SEQ, DIM), lambda r: (0, 0)),
            pl.BlockSpec((PSEQ, DIM), lambda r: (0, 0)),
        ],
        out_specs=pl.BlockSpec((RSS, DIM), lambda r: (r + r // 7, 0)),
        out_shape=jax.ShapeDtypeStruct((PSEQ, DIM), jnp.float32),
        interpret=_INTERPRET,
    )(idx, q, k, v)


def _fin_kernel(v0_ref, v1_ref, v2_ref, v3_ref, at_ref, w_ref, bl_ref,
                wo_ref, bo_ref, o_ref):
    a = pl.program_id(0)
    vp = jnp.concatenate([v0_ref[:], v1_ref[:], v2_ref[:], v3_ref[:]], axis=0)
    vpd = {du: vp[8 + du:8 + du + 7152, :] for du in (-1, 0, 1)}
    n = jax.lax.broadcasted_iota(jnp.int32, (PSLAB, 1), 0)
    h_pos = 4 * a + n // 896
    w_pos = (n // 32) % 28
    d_pos = n % 32
    hmask = {-1: (h_pos > 0), 0: None, 1: (h_pos < 27)}
    wmask = {-1: (w_pos > 0), 0: None, 1: (w_pos < 27)}
    dmask = {-1: (d_pos > 0), 0: None, 1: (d_pos < 27)}
    acc = at_ref[:] + bl_ref[:]
    for t in range(27):
        dh, dw, du = t // 9 - 1, (t // 3) % 3 - 1, t % 3 - 1
        s0 = 1784 + 896 * dh + 32 * dw
        sl = vpd[du][s0:s0 + PSLAB, :]
        m = None
        for part in (hmask[dh], wmask[dw], dmask[du]):
            if part is not None:
                m = part if m is None else (m & part)
        term = sl * w_ref[t:t + 1, :]
        if m is not None:
            term = jnp.where(m, term, 0.0)
        acc = acc + term
    o_ref[:] = jnp.dot(acc, wo_ref[:],
                       preferred_element_type=jnp.float32) + bo_ref[:]


def _fin_call(v_flat, attn_flat, w27, bl2d, wo_t, bo2d):
    return pl.pallas_call(
        _fin_kernel,
        grid=(N_WIN,),
        in_specs=[
            pl.BlockSpec((HBLK, DIM), lambda a: (jnp.clip(2 * a - 1, 0, 13), 0)),
            pl.BlockSpec((HBLK, DIM), lambda a: (2 * a, 0)),
            pl.BlockSpec((HBLK, DIM), lambda a: (2 * a + 1, 0)),
            pl.BlockSpec((HBLK, DIM), lambda a: (jnp.clip(2 * a + 2, 0, 13), 0)),
            pl.BlockSpec((PSLAB, DIM), lambda a: (a, 0)),
            pl.BlockSpec((27, DIM), lambda a: (0, 0)),
            pl.BlockSpec((1, DIM), lambda a: (0, 0)),
            pl.BlockSpec((DIM, DIM), lambda a: (0, 0)),
            pl.BlockSpec((1, DIM), lambda a: (0, 0)),
        ],
        out_specs=pl.BlockSpec((PSLAB, DIM), lambda a: (a, 0)),
        out_shape=jax.ShapeDtypeStruct((PSEQ, DIM), jnp.float32),
        interpret=_INTERPRET,
    )(v_flat, v_flat, v_flat, v_flat, attn_flat, w27, bl2d, wo_t, bo2d)


def kernel(x, W_qkv, b_qkv, W_lepe, b_lepe, W_out, b_out):
    C, H, W_, D = DIM, 28, 28, 28
    rs = H // N_WIN
    xt = x[0].reshape(C, N_WIN, rs, N_WIN, rs, N_WIN, rs)
    xt = jnp.transpose(xt, (1, 3, 5, 2, 4, 6, 0)).reshape(SEQ, C)

    q, k, v, idx = _qkvroute_call(xt, W_qkv.T, b_qkv[None, :])
    attn_seq = _attn_call(q, k, v, idx)

    def pseq2grid_flat(t):
        t = t.reshape(N_WIN, N_WIN, 8, rs, rs, rs, C)
        t = jnp.transpose(t, (0, 3, 1, 4, 2, 5, 6))
        return t.reshape(PSEQ, C)

    out_flat = _fin_call(pseq2grid_flat(v), pseq2grid_flat(attn_seq),
                         W_lepe.reshape(C, 27).T, b_lepe[None, :],
                         W_out.T, b_out[None, :])
    out = out_flat.reshape(H, W_, 32, C)[:, :, :D, :]
    return jnp.transpose(out, (3, 0, 1, 2))[None]


# speedup vs baseline: 1.3374x; 1.1789x over previous
import jax
import jax.numpy as jnp
from jax.experimental import pallas as pl
from jax.experimental.pallas import tpu as pltpu

DIM = 128
NUM_HEADS = 8
N_WIN = 7
TOPK = 4
HEAD_DIM = DIM // NUM_HEADS
SCALE = DIM ** -0.5
NREG = N_WIN ** 3
RSS = 64
SEQ = NREG * RSS
PSEQ = 7 * 7 * 8 * RSS
PSLAB = 4 * 28 * 32
HBLK = PSLAB // 2

_INTERPRET = False

_QKV_RBLK = 49


def _qkvroute_kernel(x_ref, w_ref, b_ref, q_ref, k_ref, v_ref, idx_ref,
                     pool_ref):
    i = pl.program_id(0)

    @pl.when(i < 7)
    def _qkv():
        y = jnp.dot(x_ref[:], w_ref[:],
                    preferred_element_type=jnp.float32) + b_ref[:]
        q_ref[:] = y[:, :DIM]
        k_ref[:] = y[:, DIM:2 * DIM]
        v_ref[:] = y[:, 2 * DIM:]
        p = y[:, :2 * DIM].reshape(_QKV_RBLK, RSS, 2 * DIM)
        pool_ref[i] = jnp.sum(p, axis=1) * (1.0 / RSS)

    @pl.when(i == 7)
    def _route():
        pools = pool_ref[:].reshape(NREG, 2 * DIM)
        qp = pools[:, :DIM]
        kp = pools[:, DIM:]
        a = jax.lax.dot_general(qp, kp, (((1,), (1,)), ((), ())),
                                preferred_element_type=jnp.float32)
        col = jax.lax.broadcasted_iota(jnp.int32, a.shape, 1)
        for j in range(TOPK):
            m = jnp.max(a, axis=1, keepdims=True)
            cand = jnp.where(a >= m, col, NREG + 1)
            sel = jnp.min(cand, axis=1, keepdims=True)
            idx_ref[:, j:j + 1] = sel
            a = jnp.where(col == sel, -jnp.inf, a)


def _qkvroute_call(x_seq, w_t, b2d):
    m_blk = _QKV_RBLK * RSS
    seq_spec = pl.BlockSpec((m_blk, DIM), lambda i: (jnp.clip(i, 0, 6), 0))
    return pl.pallas_call(
        _qkvroute_kernel,
        grid=(8,),
        in_specs=[
            seq_spec,
            pl.BlockSpec((DIM, 3 * DIM), lambda i: (0, 0)),
            pl.BlockSpec((1, 3 * DIM), lambda i: (0, 0)),
        ],
        out_specs=[
            seq_spec, seq_spec, seq_spec,
            pl.BlockSpec((NREG, TOPK), lambda i: (0, 0)),
        ],
        out_shape=[
            jax.ShapeDtypeStruct((SEQ, DIM), jnp.float32),
            jax.ShapeDtypeStruct((SEQ, DIM), jnp.float32),
            jax.ShapeDtypeStruct((SEQ, DIM), jnp.float32),
            jax.ShapeDtypeStruct((NREG, TOPK), jnp.int32),
        ],
        scratch_shapes=[pltpu.VMEM((7, _QKV_RBLK, 2 * DIM), jnp.float32)],
        interpret=_INTERPRET,
    )(x_seq, w_t, b2d)


def _attn_kernel(idx_ref, q_ref, k_ref, v_ref, o_ref):
    r = pl.program_id(0)
    q = q_ref[:] * SCALE
    ks, vs = [], []
    for j in range(TOPK):
        g = idx_ref[r, j]
        ks.append(k_ref[pl.ds(g * RSS, RSS), :])
        vs.append(v_ref[pl.ds(g * RSS, RSS), :])
    kg = jnp.concatenate(ks, axis=0)
    vg = jnp.concatenate(vs, axis=0)
    big = NUM_HEADS * RSS
    qt = jnp.broadcast_to(q[None], (NUM_HEADS, RSS, DIM)).reshape(big, DIM)
    rowh = jax.lax.broadcasted_iota(jnp.int32, (big, DIM), 0) // RSS
    colh = jax.lax.broadcasted_iota(jnp.int32, (big, DIM), 1) // HEAD_DIM
    qbd = jnp.where(rowh == colh, qt, 0.0).astype(jnp.bfloat16)
    s = jax.lax.dot_general(qbd, kg.astype(jnp.bfloat16),
                            (((1,), (1,)), ((), ())),
                            preferred_element_type=jnp.float32)
    s = s - jnp.max(s, axis=1, keepdims=True)
    e = jnp.exp(s)
    denom = jnp.sum(e, axis=1, keepdims=True)
    o3 = jnp.dot(e.astype(jnp.bfloat16), vg.astype(jnp.bfloat16),
                 preferred_element_type=jnp.float32)
    o3 = o3 / denom
    hsel = jax.lax.broadcasted_iota(jnp.int32, (RSS, DIM), 1) // HEAD_DIM
    acc = jnp.zeros((RSS, DIM), jnp.float32)
    for m in range(NUM_HEADS):
        acc = acc + jnp.where(hsel == m, o3[m * RSS:(m + 1) * RSS, :], 0.0)
    o_ref[:] = acc


def _attn_call(q, k, v, idx):
    return pl.pallas_call(
        _attn_kernel,
        grid=(NREG,),
        in_specs=[
            pl.BlockSpec(memory_space=pltpu.SMEM),
            pl.BlockSpec((RSS, DIM), lambda r: (r, 0)),
            pl.BlockSpec((SEQ, DIM), lambda r: (0, 0)),
            pl.BlockSpec((SEQ, DIM), lambda r: (0, 0)),
        ],
        out_specs=pl.BlockSpec((RSS, DIM), lambda r: (r, 0)),
        out_shape=jax.ShapeDtypeStruct((SEQ, DIM), jnp.float32),
        interpret=_INTERPRET,
    )(idx, q, k, v)


def _lepe_kernel(vp_ref, w_ref, b_ref, o_ref):
    acc = jnp.zeros((28, 28, 28, DIM), jnp.float32) + b_ref[:].reshape(1, 1, 1, DIM)
    for t in range(27):
        i, j, k = t // 9, (t // 3) % 3, t % 3
        w_t = w_ref[t:t + 1, :].reshape(1, 1, 1, DIM)
        acc = acc + vp_ref[i:i + 28, j:j + 28, k:k + 28, :] * w_t
    o_ref[:] = acc


def _lepe_call(v_pad, w27, b_lepe):
    return pl.pallas_call(
        _lepe_kernel,
        out_shape=jax.ShapeDtypeStruct((28, 28, 28, DIM), jnp.float32),
        interpret=_INTERPRET,
    )(v_pad, w27, b_lepe)


_PROJ_MBLK = 2744


def _proj_kernel(a_ref, l_ref, w_ref, b_ref, o_ref):
    s = a_ref[:] + l_ref[:]
    o_ref[:] = jnp.dot(s, w_ref[:], preferred_element_type=jnp.float32) + b_ref[:]


def _proj_call(attn_flat, lepe_flat, w_t, b2d):
    return pl.pallas_call(
        _proj_kernel,
        grid=(SEQ // _PROJ_MBLK,),
        in_specs=[
            pl.BlockSpec((_PROJ_MBLK, DIM), lambda i: (i, 0)),
            pl.BlockSpec((_PROJ_MBLK, DIM), lambda i: (i, 0)),
            pl.BlockSpec((DIM, DIM), lambda i: (0, 0)),
            pl.BlockSpec((1, DIM), lambda i: (0, 0)),
        ],
        out_specs=pl.BlockSpec((_PROJ_MBLK, DIM), lambda i: (i, 0)),
        out_shape=jax.ShapeDtypeStruct((SEQ, DIM), jnp.float32),
        interpret=_INTERPRET,
    )(attn_flat, lepe_flat, w_t, b2d)


def kernel(x, W_qkv, b_qkv, W_lepe, b_lepe, W_out, b_out):
    C, H, W_, D = DIM, 28, 28, 28
    rs = H // N_WIN
    xt = x[0].reshape(C, N_WIN, rs, N_WIN, rs, N_WIN, rs)
    xt = jnp.transpose(xt, (1, 3, 5, 2, 4, 6, 0)).reshape(SEQ, C)

    q, k, v, idx = _qkvroute_call(xt, W_qkv.T, b_qkv[None, :])
    attn_seq = _attn_call(q, k, v, idx)

    def seq2grid_cl(t):
        t = t.reshape(N_WIN, N_WIN, N_WIN, rs, rs, rs, C)
        t = jnp.transpose(t, (0, 3, 1, 4, 2, 5, 6))
        return t.reshape(H, W_, D, C)

    v_pad = jnp.pad(seq2grid_cl(v), ((1, 1), (1, 1), (1, 1), (0, 0)))
    lepe = _lepe_call(v_pad, W_lepe.reshape(C, 27).T, b_lepe[None, :])
    out_flat = _proj_call(seq2grid_cl(attn_seq).reshape(SEQ, C),
                          lepe.reshape(SEQ, C), W_out.T, b_out[None, :])
    out = jnp.transpose(out_flat.reshape(H, W_, D, C), (3, 0, 1, 2))
    return out[None]


# speedup vs baseline: 2.2698x; 1.6972x over previous
import jax
import jax.numpy as jnp
from jax.experimental import pallas as pl
from jax.experimental.pallas import tpu as pltpu

DIM = 128
NUM_HEADS = 8
N_WIN = 7
TOPK = 4
HEAD_DIM = DIM // NUM_HEADS
SCALE = DIM ** -0.5
NREG = N_WIN ** 3
RSS = 64
SEQ = NREG * RSS
PSEQ = 7 * 7 * 8 * RSS
PSLAB = 4 * 28 * 32
HBLK = PSLAB // 2

_INTERPRET = False

_QKV_RBLK = 49


def _qkvroute_kernel(x_ref, w_ref, b_ref, q_ref, k_ref, v_ref, idx_ref,
                     pool_ref):
    i = pl.program_id(0)

    @pl.when(i < 7)
    def _qkv():
        y = jnp.dot(x_ref[:], w_ref[:],
                    preferred_element_type=jnp.float32) + b_ref[:]
        q_ref[:] = y[:, :DIM]
        k_ref[:] = y[:, DIM:2 * DIM]
        v_ref[:] = y[:, 2 * DIM:]
        p = y[:, :2 * DIM].reshape(_QKV_RBLK, RSS, 2 * DIM)
        pool_ref[i] = jnp.sum(p, axis=1) * (1.0 / RSS)

    @pl.when(i == 7)
    def _route():
        pools = pool_ref[:].reshape(NREG, 2 * DIM)
        qp = pools[:, :DIM]
        kp = pools[:, DIM:]
        a = jax.lax.dot_general(qp, kp, (((1,), (1,)), ((), ())),
                                preferred_element_type=jnp.float32)
        col = jax.lax.broadcasted_iota(jnp.int32, a.shape, 1)
        for j in range(TOPK):
            m = jnp.max(a, axis=1, keepdims=True)
            cand = jnp.where(a >= m, col, NREG + 1)
            sel = jnp.min(cand, axis=1, keepdims=True)
            idx_ref[:, j:j + 1] = sel
            a = jnp.where(col == sel, -jnp.inf, a)


def _qkvroute_call(x_seq, w_t, b2d):
    m_blk = _QKV_RBLK * RSS
    seq_spec = pl.BlockSpec((m_blk, DIM), lambda i: (jnp.clip(i, 0, 6), 0))
    return pl.pallas_call(
        _qkvroute_kernel,
        grid=(8,),
        in_specs=[
            seq_spec,
            pl.BlockSpec((DIM, 3 * DIM), lambda i: (0, 0)),
            pl.BlockSpec((1, 3 * DIM), lambda i: (0, 0)),
        ],
        out_specs=[
            seq_spec, seq_spec, seq_spec,
            pl.BlockSpec((NREG, TOPK), lambda i: (0, 0)),
        ],
        out_shape=[
            jax.ShapeDtypeStruct((SEQ, DIM), jnp.float32),
            jax.ShapeDtypeStruct((SEQ, DIM), jnp.float32),
            jax.ShapeDtypeStruct((SEQ, DIM), jnp.float32),
            jax.ShapeDtypeStruct((NREG, TOPK), jnp.int32),
        ],
        scratch_shapes=[pltpu.VMEM((7, _QKV_RBLK, 2 * DIM), jnp.float32)],
        interpret=_INTERPRET,
    )(x_seq, w_t, b2d)


_ATT_RBLK = 7


def _attn_kernel(idx_ref, q_ref, k_ref, v_ref, o_ref):
    i = pl.program_id(0)
    big = NUM_HEADS * RSS
    rowh = jax.lax.broadcasted_iota(jnp.int32, (big, DIM), 0) // RSS
    colh = jax.lax.broadcasted_iota(jnp.int32, (big, DIM), 1) // HEAD_DIM
    band = rowh == colh
    hsel = jax.lax.broadcasted_iota(jnp.int32, (RSS, DIM), 1) // HEAD_DIM
    for t in range(_ATT_RBLK):
        r = i * _ATT_RBLK + t
        q = q_ref[t * RSS:(t + 1) * RSS, :] * SCALE
        ks, vs = [], []
        for j in range(TOPK):
            g = idx_ref[r, j]
            ks.append(k_ref[pl.ds(g * RSS, RSS), :])
            vs.append(v_ref[pl.ds(g * RSS, RSS), :])
        kg = jnp.concatenate(ks, axis=0)
        vg = jnp.concatenate(vs, axis=0)
        qt = jnp.broadcast_to(q[None], (NUM_HEADS, RSS, DIM)).reshape(big, DIM)
        qbd = jnp.where(band, qt, 0.0).astype(jnp.bfloat16)
        s = jax.lax.dot_general(qbd, kg.astype(jnp.bfloat16),
                                (((1,), (1,)), ((), ())),
                                preferred_element_type=jnp.float32)
        e = jnp.exp(s)
        denom = jnp.sum(e, axis=1, keepdims=True)
        o3 = jnp.dot(e.astype(jnp.bfloat16), vg.astype(jnp.bfloat16),
                     preferred_element_type=jnp.float32)
        o3 = o3 / denom
        acc = jnp.zeros((RSS, DIM), jnp.float32)
        for m in range(NUM_HEADS):
            acc = acc + jnp.where(hsel == m, o3[m * RSS:(m + 1) * RSS, :], 0.0)
        o_ref[t * RSS:(t + 1) * RSS, :] = acc


def _attn_call(q, k, v, idx):
    m_blk = _ATT_RBLK * RSS
    return pl.pallas_call(
        _attn_kernel,
        grid=(NREG // _ATT_RBLK,),
        in_specs=[
            pl.BlockSpec(memory_space=pltpu.SMEM),
            pl.BlockSpec((m_blk, DIM), lambda i: (i, 0)),
            pl.BlockSpec((SEQ, DIM), lambda i: (0, 0)),
            pl.BlockSpec((SEQ, DIM), lambda i: (0, 0)),
        ],
        out_specs=pl.BlockSpec((m_blk, DIM), lambda i: (i, 0)),
        out_shape=jax.ShapeDtypeStruct((SEQ, DIM), jnp.float32),
        interpret=_INTERPRET,
    )(idx, q, k, v)


def _lepe_kernel(vp_ref, w_ref, b_ref, o_ref):
    acc = jnp.zeros((28, 28, 28, DIM), jnp.float32) + b_ref[:].reshape(1, 1, 1, DIM)
    for t in range(27):
        i, j, k = t // 9, (t // 3) % 3, t % 3
        w_t = w_ref[t:t + 1, :].reshape(1, 1, 1, DIM)
        acc = acc + vp_ref[i:i + 28, j:j + 28, k:k + 28, :] * w_t
    o_ref[:] = acc


def _lepe_call(v_pad, w27, b_lepe):
    return pl.pallas_call(
        _lepe_kernel,
        out_shape=jax.ShapeDtypeStruct((28, 28, 28, DIM), jnp.float32),
        interpret=_INTERPRET,
    )(v_pad, w27, b_lepe)


_PROJ_MBLK = 2744


def _proj_kernel(a_ref, l_ref, w_ref, b_ref, o_ref):
    s = a_ref[:] + l_ref[:]
    o_ref[:] = jnp.dot(s, w_ref[:], preferred_element_type=jnp.float32) + b_ref[:]


def _proj_call(attn_flat, lepe_flat, w_t, b2d):
    return pl.pallas_call(
        _proj_kernel,
        grid=(SEQ // _PROJ_MBLK,),
        in_specs=[
            pl.BlockSpec((_PROJ_MBLK, DIM), lambda i: (i, 0)),
            pl.BlockSpec((_PROJ_MBLK, DIM), lambda i: (i, 0)),
            pl.BlockSpec((DIM, DIM), lambda i: (0, 0)),
            pl.BlockSpec((1, DIM), lambda i: (0, 0)),
        ],
        out_specs=pl.BlockSpec((_PROJ_MBLK, DIM), lambda i: (i, 0)),
        out_shape=jax.ShapeDtypeStruct((SEQ, DIM), jnp.float32),
        interpret=_INTERPRET,
    )(attn_flat, lepe_flat, w_t, b2d)


def kernel(x, W_qkv, b_qkv, W_lepe, b_lepe, W_out, b_out):
    C, H, W_, D = DIM, 28, 28, 28
    rs = H // N_WIN
    xt = x[0].reshape(C, N_WIN, rs, N_WIN, rs, N_WIN, rs)
    xt = jnp.transpose(xt, (1, 3, 5, 2, 4, 6, 0)).reshape(SEQ, C)

    q, k, v, idx = _qkvroute_call(xt, W_qkv.T, b_qkv[None, :])
    attn_seq = _attn_call(q, k, v, idx)

    def seq2grid_cl(t):
        t = t.reshape(N_WIN, N_WIN, N_WIN, rs, rs, rs, C)
        t = jnp.transpose(t, (0, 3, 1, 4, 2, 5, 6))
        return t.reshape(H, W_, D, C)

    v_pad = jnp.pad(seq2grid_cl(v), ((1, 1), (1, 1), (1, 1), (0, 0)))
    lepe = _lepe_call(v_pad, W_lepe.reshape(C, 27).T, b_lepe[None, :])
    out_flat = _proj_call(seq2grid_cl(attn_seq).reshape(SEQ, C),
                          lepe.reshape(SEQ, C), W_out.T, b_out[None, :])
    out = jnp.transpose(out_flat.reshape(H, W_, D, C), (3, 0, 1, 2))
    return out[None]


# speedup vs baseline: 2.3526x; 1.0365x over previous
import jax
import jax.numpy as jnp
from jax.experimental import pallas as pl
from jax.experimental.pallas import tpu as pltpu

DIM = 128
NUM_HEADS = 8
N_WIN = 7
TOPK = 4
HEAD_DIM = DIM // NUM_HEADS
SCALE = DIM ** -0.5
NREG = N_WIN ** 3
RSS = 64
SEQ = NREG * RSS
PSEQ = 7 * 7 * 8 * RSS
PSLAB = 4 * 28 * 32
HBLK = PSLAB // 2

_INTERPRET = False

_QKV_RBLK = 49


def _qkvroute_kernel(x_ref, w_ref, b_ref, q_ref, k_ref, v_ref, idx_ref,
                     pool_ref):
    i = pl.program_id(0)

    @pl.when(i < 7)
    def _qkv():
        y = jnp.dot(x_ref[:], w_ref[:],
                    preferred_element_type=jnp.float32) + b_ref[:]
        q_ref[:] = y[:, :DIM]
        k_ref[:] = y[:, DIM:2 * DIM]
        v_ref[:] = y[:, 2 * DIM:]
        p = y[:, :2 * DIM].reshape(_QKV_RBLK, RSS, 2 * DIM)
        pool_ref[i] = jnp.sum(p, axis=1) * (1.0 / RSS)

    @pl.when(i == 7)
    def _route():
        pools = pool_ref[:].reshape(NREG, 2 * DIM)
        qp = pools[:, :DIM]
        kp = pools[:, DIM:]
        a = jax.lax.dot_general(qp, kp, (((1,), (1,)), ((), ())),
                                preferred_element_type=jnp.float32)
        col = jax.lax.broadcasted_iota(jnp.int32, a.shape, 1)
        for j in range(TOPK):
            m = jnp.max(a, axis=1, keepdims=True)
            cand = jnp.where(a >= m, col, NREG + 1)
            sel = jnp.min(cand, axis=1, keepdims=True)
            idx_ref[:, j:j + 1] = sel
            a = jnp.where(col == sel, -jnp.inf, a)


def _qkvroute_call(x_seq, w_t, b2d):
    m_blk = _QKV_RBLK * RSS
    seq_spec = pl.BlockSpec((m_blk, DIM), lambda i: (jnp.clip(i, 0, 6), 0))
    return pl.pallas_call(
        _qkvroute_kernel,
        grid=(8,),
        in_specs=[
            seq_spec,
            pl.BlockSpec((DIM, 3 * DIM), lambda i: (0, 0)),
            pl.BlockSpec((1, 3 * DIM), lambda i: (0, 0)),
        ],
        out_specs=[
            seq_spec, seq_spec, seq_spec,
            pl.BlockSpec((NREG, TOPK), lambda i: (0, 0)),
        ],
        out_shape=[
            jax.ShapeDtypeStruct((SEQ, DIM), jnp.float32),
            jax.ShapeDtypeStruct((SEQ, DIM), jnp.float32),
            jax.ShapeDtypeStruct((SEQ, DIM), jnp.float32),
            jax.ShapeDtypeStruct((NREG, TOPK), jnp.int32),
        ],
        scratch_shapes=[pltpu.VMEM((7, _QKV_RBLK, 2 * DIM), jnp.float32)],
        interpret=_INTERPRET,
    )(x_seq, w_t, b2d)


_ATT_RBLK = 49


def _attn_kernel(idx_ref, q_ref, k_ref, v_ref, o_ref):
    i = pl.program_id(0)
    big = NUM_HEADS * RSS
    rowh = jax.lax.broadcasted_iota(jnp.int32, (big, DIM), 0) // RSS
    colh = jax.lax.broadcasted_iota(jnp.int32, (big, DIM), 1) // HEAD_DIM
    band = rowh == colh
    hsel = jax.lax.broadcasted_iota(jnp.int32, (RSS, DIM), 1) // HEAD_DIM
    for t in range(_ATT_RBLK):
        r = i * _ATT_RBLK + t
        q = q_ref[t * RSS:(t + 1) * RSS, :] * SCALE
        ks, vs = [], []
        for j in range(TOPK):
            g = idx_ref[r, j]
            ks.append(k_ref[pl.ds(g * RSS, RSS), :])
            vs.append(v_ref[pl.ds(g * RSS, RSS), :])
        kg = jnp.concatenate(ks, axis=0)
        vg = jnp.concatenate(vs, axis=0)
        qt = jnp.broadcast_to(q[None], (NUM_HEADS, RSS, DIM)).reshape(big, DIM)
        qbd = jnp.where(band, qt, 0.0).astype(jnp.bfloat16)
        s = jax.lax.dot_general(qbd, kg.astype(jnp.bfloat16),
                                (((1,), (1,)), ((), ())),
                                preferred_element_type=jnp.float32)
        e = jnp.exp(s)
        denom = jnp.sum(e, axis=1, keepdims=True)
        o3 = jnp.dot(e.astype(jnp.bfloat16), vg.astype(jnp.bfloat16),
                     preferred_element_type=jnp.float32)
        o3 = o3 / denom
        acc = jnp.zeros((RSS, DIM), jnp.float32)
        for m in range(NUM_HEADS):
            acc = acc + jnp.where(hsel == m, o3[m * RSS:(m + 1) * RSS, :], 0.0)
        o_ref[t * RSS:(t + 1) * RSS, :] = acc


def _attn_call(q, k, v, idx):
    m_blk = _ATT_RBLK * RSS
    return pl.pallas_call(
        _attn_kernel,
        grid=(NREG // _ATT_RBLK,),
        in_specs=[
            pl.BlockSpec(memory_space=pltpu.SMEM),
            pl.BlockSpec((m_blk, DIM), lambda i: (i, 0)),
            pl.BlockSpec((SEQ, DIM), lambda i: (0, 0)),
            pl.BlockSpec((SEQ, DIM), lambda i: (0, 0)),
        ],
        out_specs=pl.BlockSpec((m_blk, DIM), lambda i: (i, 0)),
        out_shape=jax.ShapeDtypeStruct((SEQ, DIM), jnp.float32),
        interpret=_INTERPRET,
    )(idx, q, k, v)


def _lepe_kernel(vp_ref, w_ref, b_ref, o_ref):
    acc = jnp.zeros((28, 28, 28, DIM), jnp.float32) + b_ref[:].reshape(1, 1, 1, DIM)
    for t in range(27):
        i, j, k = t // 9, (t // 3) % 3, t % 3
        w_t = w_ref[t:t + 1, :].reshape(1, 1, 1, DIM)
        acc = acc + vp_ref[i:i + 28, j:j + 28, k:k + 28, :] * w_t
    o_ref[:] = acc


def _lepe_call(v_pad, w27, b_lepe):
    return pl.pallas_call(
        _lepe_kernel,
        out_shape=jax.ShapeDtypeStruct((28, 28, 28, DIM), jnp.float32),
        interpret=_INTERPRET,
    )(v_pad, w27, b_lepe)


_PROJ_MBLK = 2744


def _proj_kernel(a_ref, l_ref, w_ref, b_ref, o_ref):
    s = a_ref[:] + l_ref[:]
    o_ref[:] = jnp.dot(s, w_ref[:], preferred_element_type=jnp.float32) + b_ref[:]


def _proj_call(attn_flat, lepe_flat, w_t, b2d):
    return pl.pallas_call(
        _proj_kernel,
        grid=(SEQ // _PROJ_MBLK,),
        in_specs=[
            pl.BlockSpec((_PROJ_MBLK, DIM), lambda i: (i, 0)),
            pl.BlockSpec((_PROJ_MBLK, DIM), lambda i: (i, 0)),
            pl.BlockSpec((DIM, DIM), lambda i: (0, 0)),
            pl.BlockSpec((1, DIM), lambda i: (0, 0)),
        ],
        out_specs=pl.BlockSpec((_PROJ_MBLK, DIM), lambda i: (i, 0)),
        out_shape=jax.ShapeDtypeStruct((SEQ, DIM), jnp.float32),
        interpret=_INTERPRET,
    )(attn_flat, lepe_flat, w_t, b2d)


def kernel(x, W_qkv, b_qkv, W_lepe, b_lepe, W_out, b_out):
    C, H, W_, D = DIM, 28, 28, 28
    rs = H // N_WIN
    xt = x[0].reshape(C, N_WIN, rs, N_WIN, rs, N_WIN, rs)
    xt = jnp.transpose(xt, (1, 3, 5, 2, 4, 6, 0)).reshape(SEQ, C)

    q, k, v, idx = _qkvroute_call(xt, W_qkv.T, b_qkv[None, :])
    attn_seq = _attn_call(q, k, v, idx)

    def seq2grid_cl(t):
        t = t.reshape(N_WIN, N_WIN, N_WIN, rs, rs, rs, C)
        t = jnp.transpose(t, (0, 3, 1, 4, 2, 5, 6))
        return t.reshape(H, W_, D, C)

    v_pad = jnp.pad(seq2grid_cl(v), ((1, 1), (1, 1), (1, 1), (0, 0)))
    lepe = _lepe_call(v_pad, W_lepe.reshape(C, 27).T, b_lepe[None, :])
    out_flat = _proj_call(seq2grid_cl(attn_seq).reshape(SEQ, C),
                          lepe.reshape(SEQ, C), W_out.T, b_out[None, :])
    out = jnp.transpose(out_flat.reshape(H, W_, D, C), (3, 0, 1, 2))
    return out[None]
